# baseline (speedup 1.0000x reference)
import jax
import jax.numpy as jnp
from jax import lax
from jax.experimental import pallas as pl
from jax.experimental.pallas import tpu as pltpu
from jax.experimental.pallas import tpu_sc as plsc

N = 100000
E = 3200000
EHAT = E + N
NEG = -1e30

NC = 2
NS = 16

NT = 100352
STRIPE = NT // NS
CBD = 256
CBP = 192
EP = 32 * 1024 * 102
CHUNKS_D = EP // 32 // CBD
PER_TILE_P = EP // NS
CHUNKS_P = PER_TILE_P // CBP

BLK1 = 1568
GRID1 = NT // BLK1
BLK3 = 5000
GRID3 = N // BLK3


def _k1_body(x_ref, w_ref, ps_ref, pd_ref, tabA_ref, tabB_ref, bs_ref, bd_ref):
    i = pl.program_id(0)
    xb = x_ref[...]
    hb = jnp.dot(xb, w_ref[...], preferred_element_type=jnp.float32)
    asb = jnp.dot(hb, ps_ref[...], preferred_element_type=jnp.float32)
    adb = jnp.dot(hb, pd_ref[...], preferred_element_type=jnp.float32)
    rows = i * BLK1 + lax.broadcasted_iota(jnp.int32, (BLK1, 1), 0)
    mask = rows < N
    asb = jnp.where(mask, asb, NEG)
    adb = jnp.where(mask, adb, NEG)
    piecesA = []
    for p in range(4):
        s0 = jnp.broadcast_to(asb[:, 2 * p:2 * p + 1], (BLK1, 8))
        s1 = jnp.broadcast_to(asb[:, 2 * p + 1:2 * p + 2], (BLK1, 8))
        piecesA += [hb[:, 16 * p:16 * (p + 1)], s0, s1]
    tabA_ref[...] = jnp.concatenate(piecesA, axis=1)
    piecesB = []
    for p in range(4):
        d0 = jnp.broadcast_to(adb[:, 2 * p:2 * p + 1], (BLK1, 8))
        d1 = jnp.broadcast_to(adb[:, 2 * p + 1:2 * p + 2], (BLK1, 8))
        piecesB += [d0, d1]
    zpad = jnp.zeros((BLK1, 8), jnp.float32)
    piecesB += [asb, zpad, adb, zpad, jnp.zeros((BLK1, 32), jnp.float32)]
    tabB_ref[...] = jnp.concatenate(piecesB, axis=1)
    ms = jnp.max(asb, axis=0, keepdims=True)
    md = jnp.max(adb, axis=0, keepdims=True)

    @pl.when(i == 0)
    def _():
        bs_ref[...] = ms
        bd_ref[...] = md

    @pl.when(i > 0)
    def _():
        bs_ref[...] = jnp.maximum(bs_ref[...], ms)
        bd_ref[...] = jnp.maximum(bd_ref[...], md)


def _k1(xpad, W, Ps, Pd):
    return pl.pallas_call(
        _k1_body,
        grid=(GRID1,),
        in_specs=[
            pl.BlockSpec((BLK1, 16), lambda i: (i, 0)),
            pl.BlockSpec((16, 64), lambda i: (0, 0)),
            pl.BlockSpec((64, 8), lambda i: (0, 0)),
            pl.BlockSpec((64, 8), lambda i: (0, 0)),
        ],
        out_specs=[
            pl.BlockSpec((BLK1, 128), lambda i: (i, 0)),
            pl.BlockSpec((BLK1, 128), lambda i: (i, 0)),
            pl.BlockSpec((1, 8), lambda i: (0, 0)),
            pl.BlockSpec((1, 8), lambda i: (0, 0)),
        ],
        out_shape=[
            jax.ShapeDtypeStruct((NT, 128), jnp.float32),
            jax.ShapeDtypeStruct((NT, 128), jnp.float32),
            jax.ShapeDtypeStruct((1, 8), jnp.float32),
            jax.ShapeDtypeStruct((1, 8), jnp.float32),
        ],
    )(xpad, W, Ps, Pd)


_MESH = plsc.VectorSubcoreMesh(core_axis_name="c", subcore_axis_name="s")


def _k2d_body(srcp, dstp, t16, bcfg, zeros16, den16,
              accum, srcb0, srcb1, dstb0, dstb1, adjd0, adjd1, dstS0, dstS1,
              sbuf0, sbuf1, dbuf0, dbuf1, mbuf0, mbuf1, bbuf,
              si0, si1, sg0, sg1, ss0, ss1):
    c = lax.axis_index("c")
    s = lax.axis_index("s")
    wid = s * NC + c
    srcb = (srcb0, srcb1)
    dstb = (dstb0, dstb1)
    adjd = (adjd0, adjd1)
    dstS = (dstS0, dstS1)
    sbuf = (sbuf0, sbuf1)
    dbuf = (dbuf0, dbuf1)
    mbuf = (mbuf0, mbuf1)
    si = (si0, si1)
    sg = (sg0, sg1)
    ss = (ss0, ss1)

    pltpu.sync_copy(bcfg, bbuf)
    ball = bbuf[0]
    pltpu.sync_copy(zeros16, accum.at[pl.ds(s * STRIPE, STRIPE)])
    plsc.subcore_barrier()

    base = wid * (CHUNKS_D * CBD)

    def fire_idx(k, o):
        pltpu.async_copy(srcp.at[pl.ds(o, CBD)], srcb[k], si[k])
        pltpu.async_copy(dstp.at[pl.ds(o, CBD)], dstb[k], si[k])

    def wait_idx(k):
        pltpu.make_async_copy(srcp.at[pl.ds(0, CBD)], srcb[k], si[k]).wait()
        pltpu.make_async_copy(dstp.at[pl.ds(0, CBD)], dstb[k], si[k]).wait()

    def do_adj(k):
        @plsc.parallel_loop(0, CBD // 16, unroll=4)
        def _(j):
            colo = j * 16
            dv = dstb[k][pl.ds(colo, 16)]
            adjd[k][pl.ds(colo, 16)] = dv * 8 + 5
            dstS[k][pl.ds(colo, 16)] = dv
            srcb[k][pl.ds(colo, 16)] = srcb[k][pl.ds(colo, 16)] * 8 + 4

    def fire_g(k):
        pltpu.async_copy(t16.at[srcb[k]], sbuf[k], sg[k])
        pltpu.async_copy(t16.at[adjd[k]], dbuf[k], sg[k])

    def wait_g(k):
        pltpu.make_async_copy(t16.at[srcb[k]], sbuf[k], sg[k]).wait()
        pltpu.make_async_copy(t16.at[adjd[k]], dbuf[k], sg[k]).wait()

    def compute(k):
        @plsc.parallel_loop(0, CBD, unroll=8)
        def _(e):
            ev = sbuf[k][e] + dbuf[k][e]
            ev = jnp.maximum(ev, 0.2 * ev)
            mbuf[k][e] = jnp.exp(ev - ball)

    def fire_s(k):
        pltpu.async_copy(mbuf[k], accum.at[dstS[k]], ss[k], add=True)

    def wait_s(k):
        pltpu.make_async_copy(mbuf[k], accum.at[dstS[k]], ss[k]).wait()

    fire_idx(0, base)
    wait_idx(0)
    do_adj(0)
    fire_g(0)
    fire_idx(1, base + CBD)

    def body(g2, _):
        for k in (0, 1):
            g = g2 * 2 + k
            nk = 1 - k

            @pl.when(g + 1 < CHUNKS_D)
            def _():
                wait_idx(nk)

            @pl.when(g >= 1)
            def _():
                wait_s(nk)

            @pl.when(g + 1 < CHUNKS_D)
            def _():
                do_adj(nk)
                fire_g(nk)

            wait_g(k)

            @pl.when(g + 2 < CHUNKS_D)
            def _():
                fire_idx(k, base + (g + 2) * CBD)

            compute(k)
            fire_s(k)
        return ()

    lax.fori_loop(0, CHUNKS_D // 2, body, ())
    wait_s(1)
    plsc.subcore_barrier()
    pltpu.sync_copy(accum.at[pl.ds(s * STRIPE, STRIPE)],
                    den16.at[c, pl.ds(s * STRIPE, STRIPE)])
    plsc.subcore_barrier()


def _k2d(srcp, dstp, t16, bcfg, zeros16):
    return pl.kernel(
        _k2d_body,
        out_type=jax.ShapeDtypeStruct((2, NT, 16), jnp.float32),
        mesh=_MESH,
        compiler_params=pltpu.CompilerParams(use_tc_tiling_on_sc=False),
        scratch_types=[
            pltpu.VMEM_SHARED((NT, 16), jnp.float32),
            pltpu.VMEM((CBD,), jnp.int32),
            pltpu.VMEM((CBD,), jnp.int32),
            pltpu.VMEM((CBD,), jnp.int32),
            pltpu.VMEM((CBD,), jnp.int32),
            pltpu.VMEM((CBD,), jnp.int32),
            pltpu.VMEM((CBD,), jnp.int32),
            pltpu.VMEM((CBD,), jnp.int32),
            pltpu.VMEM((CBD,), jnp.int32),
            pltpu.VMEM((CBD, 16), jnp.float32),
            pltpu.VMEM((CBD, 16), jnp.float32),
            pltpu.VMEM((CBD, 16), jnp.float32),
            pltpu.VMEM((CBD, 16), jnp.float32),
            pltpu.VMEM((CBD, 16), jnp.float32),
            pltpu.VMEM((CBD, 16), jnp.float32),
            pltpu.VMEM((8, 16), jnp.float32),
            pltpu.SemaphoreType.DMA,
            pltpu.SemaphoreType.DMA,
            pltpu.SemaphoreType.DMA,
            pltpu.SemaphoreType.DMA,
            pltpu.SemaphoreType.DMA,
            pltpu.SemaphoreType.DMA,
        ],
    )(srcp, dstp, t16, bcfg, zeros16)


def _k2p_body(srcp, dstp, t32, t16, bcfg, zeros16, msgsT,
              accum, srcb0, srcb1, dstb0, dstb1, adjd0, adjd1, dstS0, dstS1,
              g320, g321, dbuf0, dbuf1, mbuf0, mbuf1, bbuf,
              si0, si1, sg0, sg1, ss0, ss1):
    c = lax.axis_index("c")
    s = lax.axis_index("s")
    srcb = (srcb0, srcb1)
    dstb = (dstb0, dstb1)
    adjd = (adjd0, adjd1)
    dstS = (dstS0, dstS1)
    g32 = (g320, g321)
    dbuf = (dbuf0, dbuf1)
    mbuf = (mbuf0, mbuf1)
    si = (si0, si1)
    sg = (sg0, sg1)
    ss = (ss0, ss1)

    pltpu.sync_copy(bcfg, bbuf)

    for pas in range(2):
        P = 2 * c + pas
        offP = P * NT
        bp = jnp.where(c == 0, bbuf[1 + pas], bbuf[3 + pas])

        pltpu.sync_copy(zeros16, accum.at[pl.ds(s * STRIPE, STRIPE)])
        plsc.subcore_barrier()

        base = s * PER_TILE_P

        def fire_idx(k, o):
            pltpu.async_copy(srcp.at[pl.ds(o, CBP)], srcb[k], si[k])
            pltpu.async_copy(dstp.at[pl.ds(o, CBP)], dstb[k], si[k])

        def wait_idx(k):
            pltpu.make_async_copy(srcp.at[pl.ds(0, CBP)], srcb[k], si[k]).wait()
            pltpu.make_async_copy(dstp.at[pl.ds(0, CBP)], dstb[k], si[k]).wait()

        def do_adj(k):
            @plsc.parallel_loop(0, CBP // 16, unroll=4)
            def _(j):
                colo = j * 16
                dv = dstb[k][pl.ds(colo, 16)]
                adjd[k][pl.ds(colo, 16)] = dv * 8 + P
                dstS[k][pl.ds(colo, 16)] = dv
                srcb[k][pl.ds(colo, 16)] = srcb[k][pl.ds(colo, 16)] * 4 + P

        def fire_g(k):
            pltpu.async_copy(t32.at[srcb[k]], g32[k], sg[k])
            pltpu.async_copy(t16.at[adjd[k]], dbuf[k], sg[k])

        def wait_g(k):
            pltpu.make_async_copy(t32.at[srcb[k]], g32[k], sg[k]).wait()
            pltpu.make_async_copy(t16.at[adjd[k]], dbuf[k], sg[k]).wait()

        def compute(k):
            @plsc.parallel_loop(0, CBP, unroll=8)
            def _(e):
                ev = g32[k][e, pl.ds(16, 16)] + dbuf[k][e]
                ev = jnp.maximum(ev, 0.2 * ev)
                zv = jnp.exp(ev - bp)
                mbuf[k][e] = g32[k][e, pl.ds(0, 16)] * zv

        def fire_s(k):
            pltpu.async_copy(mbuf[k], accum.at[dstS[k]], ss[k], add=True)

        def wait_s(k):
            pltpu.make_async_copy(mbuf[k], accum.at[dstS[k]], ss[k]).wait()

        fire_idx(0, base)
        wait_idx(0)
        do_adj(0)
        fire_g(0)
        fire_idx(1, base + CBP)

        def body(g2, _):
            for k in (0, 1):
                g = g2 * 2 + k
                nk = 1 - k

                @pl.when(g + 1 < CHUNKS_P)
                def _():
                    wait_idx(nk)

                @pl.when(g >= 1)
                def _():
                    wait_s(nk)

                @pl.when(g + 1 < CHUNKS_P)
                def _():
                    do_adj(nk)
                    fire_g(nk)

                wait_g(k)

                @pl.when(g + 2 < CHUNKS_P)
                def _():
                    fire_idx(k, base + (g + 2) * CBP)

                compute(k)
                fire_s(k)
            return ()

        lax.fori_loop(0, CHUNKS_P // 2, body, ())
        wait_s(1)
        plsc.subcore_barrier()
        pltpu.sync_copy(accum.at[pl.ds(s * STRIPE, STRIPE)],
                        msgsT.at[P, pl.ds(s * STRIPE, STRIPE)])
        plsc.subcore_barrier()


def _k2p(srcp, dstp, t32, t16, bcfg, zeros16):
    return pl.kernel(
        _k2p_body,
        out_type=jax.ShapeDtypeStruct((4, NT, 16), jnp.float32),
        mesh=_MESH,
        compiler_params=pltpu.CompilerParams(use_tc_tiling_on_sc=False),
        scratch_types=[
            pltpu.VMEM_SHARED((NT, 16), jnp.float32),
            pltpu.VMEM((CBP,), jnp.int32),
            pltpu.VMEM((CBP,), jnp.int32),
            pltpu.VMEM((CBP,), jnp.int32),
            pltpu.VMEM((CBP,), jnp.int32),
            pltpu.VMEM((CBP,), jnp.int32),
            pltpu.VMEM((CBP,), jnp.int32),
            pltpu.VMEM((CBP,), jnp.int32),
            pltpu.VMEM((CBP,), jnp.int32),
            pltpu.VMEM((CBP, 32), jnp.float32),
            pltpu.VMEM((CBP, 32), jnp.float32),
            pltpu.VMEM((CBP, 16), jnp.float32),
            pltpu.VMEM((CBP, 16), jnp.float32),
            pltpu.VMEM((CBP, 16), jnp.float32),
            pltpu.VMEM((CBP, 16), jnp.float32),
            pltpu.VMEM((8, 16), jnp.float32),
            pltpu.SemaphoreType.DMA,
            pltpu.SemaphoreType.DMA,
            pltpu.SemaphoreType.DMA,
            pltpu.SemaphoreType.DMA,
            pltpu.SemaphoreType.DMA,
            pltpu.SemaphoreType.DMA,
        ],
    )(srcp, dstp, t32, t16, bcfg, zeros16)


def _k3_body(m_ref, d_ref, b_ref, o_ref):
    d8 = d_ref[0] + d_ref[1]
    pieces = []
    for P in range(4):
        m = m_ref[P]
        for hh in range(2):
            mm = m[:, 8 * hh:8 * (hh + 1)]
            dd = d8[:, 2 * P + hh:2 * P + hh + 1] + 1e-16
            pieces.append(mm / dd)
    v = jnp.concatenate(pieces, axis=1) + b_ref[...]
    o_ref[...] = jnp.where(v > 0, v, jnp.exp(v) - 1.0)


def _k3(msgsT, den16, bias2d):
    return pl.pallas_call(
        _k3_body,
        grid=(GRID3,),
        in_specs=[
            pl.BlockSpec((4, BLK3, 16), lambda i: (0, i, 0)),
            pl.BlockSpec((2, BLK3, 16), lambda i: (0, i, 0)),
            pl.BlockSpec((1, 64), lambda i: (0, 0)),
        ],
        out_specs=pl.BlockSpec((BLK3, 64), lambda i: (i, 0)),
        out_shape=jax.ShapeDtypeStruct((N, 64), jnp.float32),
    )(msgsT, den16, bias2d)


def kernel(x, edge_index, W, att_src, att_dst, bias):
    src = edge_index[0]
    dst = edge_index[1]
    loops = jnp.arange(N, dtype=jnp.int32)
    pad = jnp.full((EP - EHAT,), N, jnp.int32)
    srcp = jnp.concatenate([src, loops, pad])
    dstp = jnp.concatenate([dst, loops, pad])

    xpad = jnp.concatenate([x, jnp.zeros((NT - N, 16), jnp.float32)])
    eye8 = jnp.eye(8, dtype=jnp.float32)
    Ps = (att_src[:, :, None] * eye8[:, None, :]).reshape(64, 8)
    Pd = (att_dst[:, :, None] * eye8[:, None, :]).reshape(64, 8)

    tabA, tabB, bs, bd = _k1(xpad, W, Ps, Pd)

    b = bs[0] + bd[0]
    ball = jnp.concatenate([b, jnp.full((8,), 100.0, jnp.float32)])
    brows = [ball] + [
        jnp.concatenate([jnp.full((8,), b[2 * P], jnp.float32),
                         jnp.full((8,), b[2 * P + 1], jnp.float32)])
        for P in range(4)
    ] + [jnp.zeros((16,), jnp.float32)] * 3
    bcfg = jnp.stack(brows)

    zeros16 = jnp.zeros((STRIPE, 16), jnp.float32)
    t32 = tabA.reshape(4 * NT, 32)
    t16 = tabB.reshape(8 * NT, 16)
    den16 = _k2d(srcp, dstp, t16, bcfg, zeros16)
    msgsT = _k2p(srcp, dstp, t32, t16, bcfg, zeros16)

    return _k3(msgsT, den16, bias.reshape(1, 64))


# speedup vs baseline: 1.1322x; 1.1322x over previous
import jax
import jax.numpy as jnp
from jax import lax
from jax.experimental import pallas as pl
from jax.experimental.pallas import tpu as pltpu
from jax.experimental.pallas import tpu_sc as plsc

N = 100000
E = 3200000
EHAT = E + N
NEG = -1e30

NC = 2
NS = 16

NT = 100352
STRIPE = NT // NS
CBD = 256
CBP = 192
EP = 32 * 1024 * 102
CHUNKS_D = EP // 32 // CBD
PER_TILE_P = EP // NS
CHUNKS_P = PER_TILE_P // CBP

BLK1 = 1568
GRID1 = NT // BLK1
BLK3 = 5000
GRID3 = N // BLK3


def _k1_body(x_ref, w_ref, ps_ref, pd_ref,
             hs_ref, ad_ref, asd_ref, bs_ref, bd_ref):
    i = pl.program_id(0)
    xb = x_ref[...]
    hb = jnp.dot(xb, w_ref[...], preferred_element_type=jnp.float32)
    asb = jnp.dot(hb, ps_ref[...], preferred_element_type=jnp.float32)
    adb = jnp.dot(hb, pd_ref[...], preferred_element_type=jnp.float32)
    rows = i * BLK1 + lax.broadcasted_iota(jnp.int32, (BLK1, 1), 0)
    mask = rows < N
    asb = jnp.where(mask, asb, NEG)
    adb = jnp.where(mask, adb, NEG)
    for p in range(4):
        s0 = jnp.broadcast_to(asb[:, 2 * p:2 * p + 1], (BLK1, 8))
        s1 = jnp.broadcast_to(asb[:, 2 * p + 1:2 * p + 2], (BLK1, 8))
        hs_ref[p] = jnp.concatenate([hb[:, 16 * p:16 * (p + 1)], s0, s1], axis=1)
        d0 = jnp.broadcast_to(adb[:, 2 * p:2 * p + 1], (BLK1, 8))
        d1 = jnp.broadcast_to(adb[:, 2 * p + 1:2 * p + 2], (BLK1, 8))
        ad_ref[p] = jnp.concatenate([d0, d1], axis=1)
    zpad = jnp.zeros((BLK1, 8), jnp.float32)
    asd_ref[0] = jnp.concatenate([asb, zpad], axis=1)
    asd_ref[1] = jnp.concatenate([adb, zpad], axis=1)
    ms = jnp.max(asb, axis=0, keepdims=True)
    md = jnp.max(adb, axis=0, keepdims=True)

    @pl.when(i == 0)
    def _():
        bs_ref[...] = ms
        bd_ref[...] = md

    @pl.when(i > 0)
    def _():
        bs_ref[...] = jnp.maximum(bs_ref[...], ms)
        bd_ref[...] = jnp.maximum(bd_ref[...], md)


def _k1(xpad, W, Ps, Pd):
    return pl.pallas_call(
        _k1_body,
        grid=(GRID1,),
        in_specs=[
            pl.BlockSpec((BLK1, 16), lambda i: (i, 0)),
            pl.BlockSpec((16, 64), lambda i: (0, 0)),
            pl.BlockSpec((64, 8), lambda i: (0, 0)),
            pl.BlockSpec((64, 8), lambda i: (0, 0)),
        ],
        out_specs=[
            pl.BlockSpec((4, BLK1, 32), lambda i: (0, i, 0)),
            pl.BlockSpec((4, BLK1, 16), lambda i: (0, i, 0)),
            pl.BlockSpec((2, BLK1, 16), lambda i: (0, i, 0)),
            pl.BlockSpec((1, 8), lambda i: (0, 0)),
            pl.BlockSpec((1, 8), lambda i: (0, 0)),
        ],
        out_shape=[
            jax.ShapeDtypeStruct((4, NT, 32), jnp.float32),
            jax.ShapeDtypeStruct((4, NT, 16), jnp.float32),
            jax.ShapeDtypeStruct((2, NT, 16), jnp.float32),
            jax.ShapeDtypeStruct((1, 8), jnp.float32),
            jax.ShapeDtypeStruct((1, 8), jnp.float32),
        ],
    )(xpad, W, Ps, Pd)


_MESH = plsc.VectorSubcoreMesh(core_axis_name="c", subcore_axis_name="s")


def _k2d_body(srcp, dstp, t16, bcfg, zeros16, den16,
              accum, srcb0, srcb1, dstb0, dstb1, adjd0, adjd1, dstS0, dstS1,
              sbuf0, sbuf1, dbuf0, dbuf1, mbuf0, mbuf1, bbuf,
              si0, si1, sg0, sg1, ss0, ss1):
    c = lax.axis_index("c")
    s = lax.axis_index("s")
    wid = s * NC + c
    srcb = (srcb0, srcb1)
    dstb = (dstb0, dstb1)
    adjd = (adjd0, adjd1)
    dstS = (dstS0, dstS1)
    sbuf = (sbuf0, sbuf1)
    dbuf = (dbuf0, dbuf1)
    mbuf = (mbuf0, mbuf1)
    si = (si0, si1)
    sg = (sg0, sg1)
    ss = (ss0, ss1)

    pltpu.sync_copy(bcfg, bbuf)
    ball = bbuf[0]
    pltpu.sync_copy(zeros16, accum.at[pl.ds(s * STRIPE, STRIPE)])
    plsc.subcore_barrier()

    base = wid * (CHUNKS_D * CBD)

    def fire_idx(k, o):
        pltpu.async_copy(srcp.at[pl.ds(o, CBD)], srcb[k], si[k])
        pltpu.async_copy(dstp.at[pl.ds(o, CBD)], dstb[k], si[k])

    def wait_idx(k):
        pltpu.make_async_copy(srcp.at[pl.ds(0, CBD)], srcb[k], si[k]).wait()
        pltpu.make_async_copy(dstp.at[pl.ds(0, CBD)], dstb[k], si[k]).wait()

    def do_adj(k):
        @plsc.parallel_loop(0, CBD // 16, unroll=4)
        def _(j):
            colo = j * 16
            dv = dstb[k][pl.ds(colo, 16)]
            adjd[k][pl.ds(colo, 16)] = dv + NT
            dstS[k][pl.ds(colo, 16)] = dv

    def fire_g(k):
        pltpu.async_copy(t16.at[srcb[k]], sbuf[k], sg[k])
        pltpu.async_copy(t16.at[adjd[k]], dbuf[k], sg[k])

    def wait_g(k):
        pltpu.make_async_copy(t16.at[srcb[k]], sbuf[k], sg[k]).wait()
        pltpu.make_async_copy(t16.at[adjd[k]], dbuf[k], sg[k]).wait()

    def compute(k):
        @plsc.parallel_loop(0, CBD, unroll=8)
        def _(e):
            ev = sbuf[k][e] + dbuf[k][e]
            ev = jnp.maximum(ev, 0.2 * ev)
            mbuf[k][e] = jnp.exp(ev - ball)

    def fire_s(k):
        pltpu.async_copy(mbuf[k], accum.at[dstS[k]], ss[k], add=True)

    def wait_s(k):
        pltpu.make_async_copy(mbuf[k], accum.at[dstS[k]], ss[k]).wait()

    fire_idx(0, base)
    wait_idx(0)
    do_adj(0)
    fire_g(0)
    fire_idx(1, base + CBD)

    def body(g2, _):
        for k in (0, 1):
            g = g2 * 2 + k
            nk = 1 - k

            @pl.when(g + 1 < CHUNKS_D)
            def _():
                wait_idx(nk)

            @pl.when(g >= 1)
            def _():
                wait_s(nk)

            @pl.when(g + 1 < CHUNKS_D)
            def _():
                do_adj(nk)
                fire_g(nk)

            wait_g(k)

            @pl.when(g + 2 < CHUNKS_D)
            def _():
                fire_idx(k, base + (g + 2) * CBD)

            compute(k)
            fire_s(k)
        return ()

    lax.fori_loop(0, CHUNKS_D // 2, body, ())
    wait_s(1)
    plsc.subcore_barrier()
    pltpu.sync_copy(accum.at[pl.ds(s * STRIPE, STRIPE)],
                    den16.at[c, pl.ds(s * STRIPE, STRIPE)])
    plsc.subcore_barrier()


def _k2d(srcp, dstp, t16, bcfg, zeros16):
    return pl.kernel(
        _k2d_body,
        out_type=jax.ShapeDtypeStruct((2, NT, 16), jnp.float32),
        mesh=_MESH,
        compiler_params=pltpu.CompilerParams(use_tc_tiling_on_sc=False),
        scratch_types=[
            pltpu.VMEM_SHARED((NT, 16), jnp.float32),
            pltpu.VMEM((CBD,), jnp.int32),
            pltpu.VMEM((CBD,), jnp.int32),
            pltpu.VMEM((CBD,), jnp.int32),
            pltpu.VMEM((CBD,), jnp.int32),
            pltpu.VMEM((CBD,), jnp.int32),
            pltpu.VMEM((CBD,), jnp.int32),
            pltpu.VMEM((CBD,), jnp.int32),
            pltpu.VMEM((CBD,), jnp.int32),
            pltpu.VMEM((CBD, 16), jnp.float32),
            pltpu.VMEM((CBD, 16), jnp.float32),
            pltpu.VMEM((CBD, 16), jnp.float32),
            pltpu.VMEM((CBD, 16), jnp.float32),
            pltpu.VMEM((CBD, 16), jnp.float32),
            pltpu.VMEM((CBD, 16), jnp.float32),
            pltpu.VMEM((8, 16), jnp.float32),
            pltpu.SemaphoreType.DMA,
            pltpu.SemaphoreType.DMA,
            pltpu.SemaphoreType.DMA,
            pltpu.SemaphoreType.DMA,
            pltpu.SemaphoreType.DMA,
            pltpu.SemaphoreType.DMA,
        ],
    )(srcp, dstp, t16, bcfg, zeros16)


def _k2p_body(srcp, dstp, t32, t16, bcfg, zeros16, msgsT,
              accum, srcb0, srcb1, dstb0, dstb1, adjd0, adjd1, dstS0, dstS1,
              g320, g321, dbuf0, dbuf1, mbuf0, mbuf1, bbuf,
              si0, si1, sg0, sg1, ss0, ss1):
    c = lax.axis_index("c")
    s = lax.axis_index("s")
    srcb = (srcb0, srcb1)
    dstb = (dstb0, dstb1)
    adjd = (adjd0, adjd1)
    dstS = (dstS0, dstS1)
    g32 = (g320, g321)
    dbuf = (dbuf0, dbuf1)
    mbuf = (mbuf0, mbuf1)
    si = (si0, si1)
    sg = (sg0, sg1)
    ss = (ss0, ss1)

    pltpu.sync_copy(bcfg, bbuf)

    for pas in range(2):
        P = 2 * c + pas
        offP = P * NT
        bp = jnp.where(c == 0, bbuf[1 + pas], bbuf[3 + pas])

        pltpu.sync_copy(zeros16, accum.at[pl.ds(s * STRIPE, STRIPE)])
        plsc.subcore_barrier()

        base = s * PER_TILE_P

        def fire_idx(k, o):
            pltpu.async_copy(srcp.at[pl.ds(o, CBP)], srcb[k], si[k])
            pltpu.async_copy(dstp.at[pl.ds(o, CBP)], dstb[k], si[k])

        def wait_idx(k):
            pltpu.make_async_copy(srcp.at[pl.ds(0, CBP)], srcb[k], si[k]).wait()
            pltpu.make_async_copy(dstp.at[pl.ds(0, CBP)], dstb[k], si[k]).wait()

        def do_adj(k):
            @plsc.parallel_loop(0, CBP // 16, unroll=4)
            def _(j):
                colo = j * 16
                dv = dstb[k][pl.ds(colo, 16)]
                adjd[k][pl.ds(colo, 16)] = dv + offP
                dstS[k][pl.ds(colo, 16)] = dv
                srcb[k][pl.ds(colo, 16)] = srcb[k][pl.ds(colo, 16)] + offP

        def fire_g(k):
            pltpu.async_copy(t32.at[srcb[k]], g32[k], sg[k])
            pltpu.async_copy(t16.at[adjd[k]], dbuf[k], sg[k])

        def wait_g(k):
            pltpu.make_async_copy(t32.at[srcb[k]], g32[k], sg[k]).wait()
            pltpu.make_async_copy(t16.at[adjd[k]], dbuf[k], sg[k]).wait()

        def compute(k):
            @plsc.parallel_loop(0, CBP, unroll=8)
            def _(e):
                ev = g32[k][e, pl.ds(16, 16)] + dbuf[k][e]
                ev = jnp.maximum(ev, 0.2 * ev)
                zv = jnp.exp(ev - bp)
                mbuf[k][e] = g32[k][e, pl.ds(0, 16)] * zv

        def fire_s(k):
            pltpu.async_copy(mbuf[k], accum.at[dstS[k]], ss[k], add=True)

        def wait_s(k):
            pltpu.make_async_copy(mbuf[k], accum.at[dstS[k]], ss[k]).wait()

        fire_idx(0, base)
        wait_idx(0)
        do_adj(0)
        fire_g(0)
        fire_idx(1, base + CBP)

        def body(g2, _):
            for k in (0, 1):
                g = g2 * 2 + k
                nk = 1 - k

                @pl.when(g + 1 < CHUNKS_P)
                def _():
                    wait_idx(nk)

                @pl.when(g >= 1)
                def _():
                    wait_s(nk)

                @pl.when(g + 1 < CHUNKS_P)
                def _():
                    do_adj(nk)
                    fire_g(nk)

                wait_g(k)

                @pl.when(g + 2 < CHUNKS_P)
                def _():
                    fire_idx(k, base + (g + 2) * CBP)

                compute(k)
                fire_s(k)
            return ()

        lax.fori_loop(0, CHUNKS_P // 2, body, ())
        wait_s(1)
        plsc.subcore_barrier()
        pltpu.sync_copy(accum.at[pl.ds(s * STRIPE, STRIPE)],
                        msgsT.at[P, pl.ds(s * STRIPE, STRIPE)])
        plsc.subcore_barrier()


def _k2p(srcp, dstp, t32, t16, bcfg, zeros16):
    return pl.kernel(
        _k2p_body,
        out_type=jax.ShapeDtypeStruct((4, NT, 16), jnp.float32),
        mesh=_MESH,
        compiler_params=pltpu.CompilerParams(use_tc_tiling_on_sc=False),
        scratch_types=[
            pltpu.VMEM_SHARED((NT, 16), jnp.float32),
            pltpu.VMEM((CBP,), jnp.int32),
            pltpu.VMEM((CBP,), jnp.int32),
            pltpu.VMEM((CBP,), jnp.int32),
            pltpu.VMEM((CBP,), jnp.int32),
            pltpu.VMEM((CBP,), jnp.int32),
            pltpu.VMEM((CBP,), jnp.int32),
            pltpu.VMEM((CBP,), jnp.int32),
            pltpu.VMEM((CBP,), jnp.int32),
            pltpu.VMEM((CBP, 32), jnp.float32),
            pltpu.VMEM((CBP, 32), jnp.float32),
            pltpu.VMEM((CBP, 16), jnp.float32),
            pltpu.VMEM((CBP, 16), jnp.float32),
            pltpu.VMEM((CBP, 16), jnp.float32),
            pltpu.VMEM((CBP, 16), jnp.float32),
            pltpu.VMEM((8, 16), jnp.float32),
            pltpu.SemaphoreType.DMA,
            pltpu.SemaphoreType.DMA,
            pltpu.SemaphoreType.DMA,
            pltpu.SemaphoreType.DMA,
            pltpu.SemaphoreType.DMA,
            pltpu.SemaphoreType.DMA,
        ],
    )(srcp, dstp, t32, t16, bcfg, zeros16)


def _k3_body(m_ref, d_ref, b_ref, o_ref):
    d8 = d_ref[0] + d_ref[1]
    pieces = []
    for P in range(4):
        m = m_ref[P]
        for hh in range(2):
            mm = m[:, 8 * hh:8 * (hh + 1)]
            dd = d8[:, 2 * P + hh:2 * P + hh + 1] + 1e-16
            pieces.append(mm / dd)
    v = jnp.concatenate(pieces, axis=1) + b_ref[...]
    o_ref[...] = jnp.where(v > 0, v, jnp.exp(v) - 1.0)


def _k3(msgsT, den16, bias2d):
    return pl.pallas_call(
        _k3_body,
        grid=(GRID3,),
        in_specs=[
            pl.BlockSpec((4, BLK3, 16), lambda i: (0, i, 0)),
            pl.BlockSpec((2, BLK3, 16), lambda i: (0, i, 0)),
            pl.BlockSpec((1, 64), lambda i: (0, 0)),
        ],
        out_specs=pl.BlockSpec((BLK3, 64), lambda i: (i, 0)),
        out_shape=jax.ShapeDtypeStruct((N, 64), jnp.float32),
    )(msgsT, den16, bias2d)


def kernel(x, edge_index, W, att_src, att_dst, bias):
    src = edge_index[0]
    dst = edge_index[1]
    loops = jnp.arange(N, dtype=jnp.int32)
    pad = jnp.full((EP - EHAT,), N, jnp.int32)
    srcp = jnp.concatenate([src, loops, pad])
    dstp = jnp.concatenate([dst, loops, pad])

    xpad = jnp.concatenate([x, jnp.zeros((NT - N, 16), jnp.float32)])
    eye8 = jnp.eye(8, dtype=jnp.float32)
    Ps = (att_src[:, :, None] * eye8[:, None, :]).reshape(64, 8)
    Pd = (att_dst[:, :, None] * eye8[:, None, :]).reshape(64, 8)

    hs32, ad16, asd, bs, bd = _k1(xpad, W, Ps, Pd)

    b = bs[0] + bd[0]
    ball = jnp.concatenate([b, jnp.full((8,), 100.0, jnp.float32)])
    brows = [ball] + [
        jnp.concatenate([jnp.full((8,), b[2 * P], jnp.float32),
                         jnp.full((8,), b[2 * P + 1], jnp.float32)])
        for P in range(4)
    ] + [jnp.zeros((16,), jnp.float32)] * 3
    bcfg = jnp.stack(brows)

    zeros16 = jnp.zeros((STRIPE, 16), jnp.float32)
    den16 = _k2d(srcp, dstp, asd.reshape(2 * NT, 16), bcfg, zeros16)
    msgsT = _k2p(srcp, dstp, hs32.reshape(4 * NT, 32),
                 ad16.reshape(4 * NT, 16), bcfg, zeros16)

    return _k3(msgsT, den16, bias.reshape(1, 64))
